# Initial kernel scaffold; baseline (speedup 1.0000x reference)
import jax
import jax.numpy as jnp
from jax.experimental import pallas as pl


def kernel(ranks, suits, rank_weight, suit_weight):
    raise NotImplementedError("write your pallas kernel here")


# speedup vs baseline: 6.1668x; 6.1668x over previous
import functools

import jax
import jax.numpy as jnp
from jax import lax
from jax.experimental import pallas as pl
from jax.experimental.pallas import tpu as pltpu
from jax.experimental.pallas import tpu_sc as plsc

NUM_WORKERS = 32
LANES = 16


def _make_sc_embed(total, out_dim, chunk):
    assert total % (NUM_WORKERS * chunk) == 0
    per_worker = total // NUM_WORKERS
    num_chunks = per_worker // chunk
    mesh = plsc.VectorSubcoreMesh(core_axis_name="c", subcore_axis_name="s")

    @functools.partial(
        pl.kernel,
        mesh=mesh,
        compiler_params=pltpu.CompilerParams(use_tc_tiling_on_sc=False),
        out_type=jax.ShapeDtypeStruct((total, out_dim), jnp.float32),
        scratch_types=[
            pltpu.VMEM((chunk,), jnp.int32),
            pltpu.VMEM((chunk,), jnp.int32),
            pltpu.VMEM((chunk,), jnp.int32),
            pltpu.VMEM((chunk, out_dim), jnp.float32),
            pltpu.SemaphoreType.DMA,
        ],
    )
    def sc_embed(ranks_hbm, suits_hbm, table_hbm, out_hbm,
                 ranks_v, suits_v, idx_v, rows_v, sem):
        wid = lax.axis_index("s") * 2 + lax.axis_index("c")
        base0 = wid * per_worker

        def chunk_body(ci, carry):
            base = base0 + ci * chunk
            pltpu.sync_copy(ranks_hbm.at[pl.ds(base, chunk)], ranks_v)
            pltpu.sync_copy(suits_hbm.at[pl.ds(base, chunk)], suits_v)

            def fuse(i, c):
                off = i * LANES
                r = ranks_v[pl.ds(off, LANES)]
                s = suits_v[pl.ds(off, LANES)]
                idx_v[pl.ds(off, LANES)] = r * 5 + s
                return c

            lax.fori_loop(0, chunk // LANES, fuse, 0, unroll=4)

            pltpu.async_copy(table_hbm.at[idx_v], rows_v, sem).wait()
            pltpu.sync_copy(rows_v, out_hbm.at[pl.ds(base, chunk)])
            return carry

        lax.fori_loop(0, num_chunks, chunk_body, 0)

    return sc_embed


_sc_embed_cached = None


def _get_sc_embed(total, out_dim, chunk):
    global _sc_embed_cached
    if _sc_embed_cached is None:
        _sc_embed_cached = _make_sc_embed(total, out_dim, chunk)
    return _sc_embed_cached


def kernel(ranks, suits, rank_weight, suit_weight):
    batch, num_cards = ranks.shape
    num_ranks, rank_dim = rank_weight.shape
    num_suits, suit_dim = suit_weight.shape
    out_dim = rank_dim + suit_dim

    combined = jnp.concatenate(
        [
            jnp.repeat(rank_weight, num_suits, axis=0),
            jnp.tile(suit_weight, (num_ranks, 1)),
        ],
        axis=1,
    )

    total = batch * num_cards
    fn = _get_sc_embed(total, out_dim, 2048)
    out = fn(ranks.reshape(total), suits.reshape(total), combined)
    return out.reshape(batch, num_cards, out_dim)


# speedup vs baseline: 6.2015x; 1.0056x over previous
import functools

import jax
import jax.numpy as jnp
from jax import lax
from jax.experimental import pallas as pl
from jax.experimental.pallas import tpu as pltpu
from jax.experimental.pallas import tpu_sc as plsc

NUM_WORKERS = 32
LANES = 16


def _make_sc_embed(total, out_dim, chunk):
    assert total % (NUM_WORKERS * chunk) == 0
    per_worker = total // NUM_WORKERS
    num_chunks = per_worker // chunk
    mesh = plsc.VectorSubcoreMesh(core_axis_name="c", subcore_axis_name="s")

    @functools.partial(
        pl.kernel,
        mesh=mesh,
        compiler_params=pltpu.CompilerParams(use_tc_tiling_on_sc=False),
        out_type=jax.ShapeDtypeStruct((total, out_dim), jnp.float32),
        scratch_types=[
            [pltpu.VMEM((chunk,), jnp.int32)] * 2,
            [pltpu.VMEM((chunk,), jnp.int32)] * 2,
            [pltpu.VMEM((chunk,), jnp.int32)] * 2,
            [pltpu.VMEM((chunk, out_dim), jnp.float32)] * 2,
            [pltpu.SemaphoreType.DMA] * 2,
            [pltpu.SemaphoreType.DMA] * 2,
        ],
    )
    def sc_embed(ranks_hbm, suits_hbm, table_hbm, out_hbm,
                 ranks_v, suits_v, idx_v, rows_v, sem_g, sem_s):
        wid = lax.axis_index("s") * 2 + lax.axis_index("c")
        base0 = wid * per_worker

        def load_fuse_gather(ci, b):
            base = base0 + ci * chunk
            pltpu.sync_copy(ranks_hbm.at[pl.ds(base, chunk)], ranks_v[b])
            pltpu.sync_copy(suits_hbm.at[pl.ds(base, chunk)], suits_v[b])

            def fuse(i, c):
                off = i * LANES
                r = ranks_v[b][pl.ds(off, LANES)]
                s = suits_v[b][pl.ds(off, LANES)]
                idx_v[b][pl.ds(off, LANES)] = r * 5 + s
                return c

            lax.fori_loop(0, chunk // LANES, fuse, 0, unroll=4)
            return pltpu.async_copy(table_hbm.at[idx_v[b]], rows_v[b], sem_g[b])

        gathers = [None, None]
        scatters = [None, None]
        gathers[0] = load_fuse_gather(0, 0)
        for ci in range(num_chunks):
            b = ci % 2
            nb = 1 - b
            if ci >= 1:
                scatters[nb].wait()
            if ci + 1 < num_chunks:
                gathers[nb] = load_fuse_gather(ci + 1, nb)
            gathers[b].wait()
            base = base0 + ci * chunk
            scatters[b] = pltpu.async_copy(
                rows_v[b], out_hbm.at[pl.ds(base, chunk)], sem_s[b])
        scatters[(num_chunks - 1) % 2].wait()

    return sc_embed


_sc_embed_cached = None


def _get_sc_embed(total, out_dim, chunk):
    global _sc_embed_cached
    if _sc_embed_cached is None:
        _sc_embed_cached = _make_sc_embed(total, out_dim, chunk)
    return _sc_embed_cached


def kernel(ranks, suits, rank_weight, suit_weight):
    batch, num_cards = ranks.shape
    num_ranks, rank_dim = rank_weight.shape
    num_suits, suit_dim = suit_weight.shape
    out_dim = rank_dim + suit_dim

    combined = jnp.concatenate(
        [
            jnp.repeat(rank_weight, num_suits, axis=0),
            jnp.tile(suit_weight, (num_ranks, 1)),
        ],
        axis=1,
    )

    total = batch * num_cards
    fn = _get_sc_embed(total, out_dim, 2048)
    out = fn(ranks.reshape(total), suits.reshape(total), combined)
    return out.reshape(batch, num_cards, out_dim)


# speedup vs baseline: 10.0552x; 1.6214x over previous
import functools

import jax
import jax.numpy as jnp
from jax import lax
from jax.experimental import pallas as pl
from jax.experimental.pallas import tpu as pltpu
from jax.experimental.pallas import tpu_sc as plsc

NUM_WORKERS = 32
LANES = 16


def _make_sc_embed(total, out_dim, chunk):
    assert total % (NUM_WORKERS * chunk) == 0
    per_worker = total // NUM_WORKERS
    num_chunks = per_worker // chunk
    mesh = plsc.VectorSubcoreMesh(core_axis_name="c", subcore_axis_name="s")

    @functools.partial(
        pl.kernel,
        mesh=mesh,
        compiler_params=pltpu.CompilerParams(use_tc_tiling_on_sc=False),
        out_type=jax.ShapeDtypeStruct((total, out_dim), jnp.float32),
        scratch_types=[
            [pltpu.VMEM((2 * chunk,), jnp.int32)] * 2,
            [pltpu.VMEM((2 * chunk,), jnp.int32)] * 2,
            [pltpu.VMEM((chunk,), jnp.int32)] * 2,
            [pltpu.VMEM((chunk, out_dim), jnp.float32)] * 2,
            [pltpu.SemaphoreType.DMA] * 2,
            [pltpu.SemaphoreType.DMA] * 2,
        ],
    )
    def sc_embed(ranks_hbm, suits_hbm, table_hbm, out_hbm,
                 ranks_v, suits_v, idx_v, rows_v, sem_g, sem_s):
        wid = lax.axis_index("s") * 2 + lax.axis_index("c")
        base0 = wid * per_worker

        def load_fuse_gather(ci, b):
            base = base0 + ci * chunk
            pltpu.sync_copy(ranks_hbm.at[pl.ds(2 * base, 2 * chunk)], ranks_v[b])
            pltpu.sync_copy(suits_hbm.at[pl.ds(2 * base, 2 * chunk)], suits_v[b])

            def fuse(i, c):
                off = i * 2 * LANES
                lane = lax.iota(jnp.int32, LANES)
                half = lane < 8
                evens = lane * 2 - jnp.where(half, 0, LANES)
                odds = evens + 1
                r0 = ranks_v[b][pl.ds(off, LANES)]
                r1 = ranks_v[b][pl.ds(off + LANES, LANES)]
                s0 = suits_v[b][pl.ds(off, LANES)]
                s1 = suits_v[b][pl.ds(off + LANES, LANES)]
                k0 = r0 * 5 + s0
                k1 = r1 * 5 + s1
                ke = jnp.where(half,
                               jnp.take_along_axis(k0, evens, axis=0),
                               jnp.take_along_axis(k1, evens, axis=0))
                ko = jnp.where(half,
                               jnp.take_along_axis(k0, odds, axis=0),
                               jnp.take_along_axis(k1, odds, axis=0))
                idx_v[b][pl.ds(i * LANES, LANES)] = ke * 70 + ko
                return c

            lax.fori_loop(0, chunk // LANES, fuse, 0, unroll=4)
            return pltpu.async_copy(table_hbm.at[idx_v[b]], rows_v[b], sem_g[b])

        gathers = [None, None]
        scatters = [None, None]
        gathers[0] = load_fuse_gather(0, 0)
        for ci in range(num_chunks):
            b = ci % 2
            nb = 1 - b
            if ci >= 1:
                scatters[nb].wait()
            if ci + 1 < num_chunks:
                gathers[nb] = load_fuse_gather(ci + 1, nb)
            gathers[b].wait()
            base = base0 + ci * chunk
            scatters[b] = pltpu.async_copy(
                rows_v[b], out_hbm.at[pl.ds(base, chunk)], sem_s[b])
        scatters[(num_chunks - 1) % 2].wait()

    return sc_embed


_sc_embed_cached = None


def _get_sc_embed(total, out_dim, chunk):
    global _sc_embed_cached
    if _sc_embed_cached is None:
        _sc_embed_cached = _make_sc_embed(total, out_dim, chunk)
    return _sc_embed_cached


def kernel(ranks, suits, rank_weight, suit_weight):
    batch, num_cards = ranks.shape
    num_ranks, rank_dim = rank_weight.shape
    num_suits, suit_dim = suit_weight.shape
    out_dim = rank_dim + suit_dim

    combined = jnp.concatenate(
        [
            jnp.repeat(rank_weight, num_suits, axis=0),
            jnp.tile(suit_weight, (num_ranks, 1)),
        ],
        axis=1,
    )
    num_fused = num_ranks * num_suits
    pair_table = jnp.concatenate(
        [
            jnp.repeat(combined, num_fused, axis=0),
            jnp.tile(combined, (num_fused, 1)),
        ],
        axis=1,
    )

    total_pairs = batch * num_cards // 2
    fn = _get_sc_embed(total_pairs, 2 * out_dim, 1024)
    out = fn(ranks.reshape(-1), suits.reshape(-1), pair_table)
    return out.reshape(batch, num_cards, out_dim)


# speedup vs baseline: 10.6853x; 1.0627x over previous
import functools

import jax
import jax.numpy as jnp
from jax import lax
from jax.experimental import pallas as pl
from jax.experimental.pallas import tpu as pltpu
from jax.experimental.pallas import tpu_sc as plsc

NUM_WORKERS = 32
LANES = 16


def _make_sc_embed(total, out_dim, chunk):
    assert total % (NUM_WORKERS * chunk) == 0
    per_worker = total // NUM_WORKERS
    num_chunks = per_worker // chunk
    mesh = plsc.VectorSubcoreMesh(core_axis_name="c", subcore_axis_name="s")

    @functools.partial(
        pl.kernel,
        mesh=mesh,
        compiler_params=pltpu.CompilerParams(use_tc_tiling_on_sc=False),
        out_type=jax.ShapeDtypeStruct((total, out_dim), jnp.float32),
        scratch_types=[
            [pltpu.VMEM((2 * chunk,), jnp.int32)] * 2,
            [pltpu.VMEM((chunk,), jnp.int32)] * 2,
            [pltpu.VMEM((chunk, out_dim), jnp.float32)] * 2,
            [pltpu.SemaphoreType.DMA] * 2,
            [pltpu.SemaphoreType.DMA] * 2,
        ],
    )
    def sc_embed(keys_hbm, table_hbm, out_hbm,
                 keys_v, idx_v, rows_v, sem_g, sem_s):
        wid = lax.axis_index("s") * 2 + lax.axis_index("c")
        base0 = wid * per_worker

        def load_fuse_gather(ci, b):
            base = base0 + ci * chunk
            pltpu.sync_copy(keys_hbm.at[pl.ds(2 * base, 2 * chunk)], keys_v[b])

            def fuse(i, c):
                off = i * 2 * LANES
                lane = lax.iota(jnp.int32, LANES)
                half = lane < 8
                evens = lane * 2 - jnp.where(half, 0, LANES)
                odds = evens + 1
                k0 = keys_v[b][pl.ds(off, LANES)]
                k1 = keys_v[b][pl.ds(off + LANES, LANES)]
                ke = jnp.where(half,
                               jnp.take_along_axis(k0, evens, axis=0),
                               jnp.take_along_axis(k1, evens, axis=0))
                ko = jnp.where(half,
                               jnp.take_along_axis(k0, odds, axis=0),
                               jnp.take_along_axis(k1, odds, axis=0))
                idx_v[b][pl.ds(i * LANES, LANES)] = ke * 70 + ko
                return c

            lax.fori_loop(0, chunk // LANES, fuse, 0, unroll=4)
            return pltpu.async_copy(table_hbm.at[idx_v[b]], rows_v[b], sem_g[b])

        gathers = [None, None]
        scatters = [None, None]
        gathers[0] = load_fuse_gather(0, 0)
        for ci in range(num_chunks):
            b = ci % 2
            nb = 1 - b
            if ci >= 1:
                scatters[nb].wait()
            if ci + 1 < num_chunks:
                gathers[nb] = load_fuse_gather(ci + 1, nb)
            gathers[b].wait()
            base = base0 + ci * chunk
            scatters[b] = pltpu.async_copy(
                rows_v[b], out_hbm.at[pl.ds(base, chunk)], sem_s[b])
        scatters[(num_chunks - 1) % 2].wait()

    return sc_embed


_sc_embed_cached = None


def _get_sc_embed(total, out_dim, chunk):
    global _sc_embed_cached
    if _sc_embed_cached is None:
        _sc_embed_cached = _make_sc_embed(total, out_dim, chunk)
    return _sc_embed_cached


def kernel(ranks, suits, rank_weight, suit_weight):
    batch, num_cards = ranks.shape
    num_ranks, rank_dim = rank_weight.shape
    num_suits, suit_dim = suit_weight.shape
    out_dim = rank_dim + suit_dim

    combined = jnp.concatenate(
        [
            jnp.repeat(rank_weight, num_suits, axis=0),
            jnp.tile(suit_weight, (num_ranks, 1)),
        ],
        axis=1,
    )
    num_fused = num_ranks * num_suits
    pair_table = jnp.concatenate(
        [
            jnp.repeat(combined, num_fused, axis=0),
            jnp.tile(combined, (num_fused, 1)),
        ],
        axis=1,
    )

    keys = (ranks * num_suits + suits).reshape(-1)

    total_pairs = batch * num_cards // 2
    fn = _get_sc_embed(total_pairs, 2 * out_dim, 1024)
    out = fn(keys, pair_table)
    return out.reshape(batch, num_cards, out_dim)


# speedup vs baseline: 19.3314x; 1.8092x over previous
import functools

import jax
import jax.numpy as jnp
from jax import lax
from jax.experimental import pallas as pl
from jax.experimental.pallas import tpu as pltpu
from jax.experimental.pallas import tpu_sc as plsc

NUM_WORKERS = 32
LANES = 16


def _make_sc_embed(batch, num_cards, out_dim, num_keys):
    assert batch % (NUM_WORKERS * LANES) == 0
    per_worker = batch // NUM_WORKERS
    mesh = plsc.VectorSubcoreMesh(core_axis_name="c", subcore_axis_name="s")

    @functools.partial(
        pl.kernel,
        mesh=mesh,
        compiler_params=pltpu.CompilerParams(
            use_tc_tiling_on_sc=False, needs_layout_passes=False),
        out_type=jax.ShapeDtypeStruct((num_cards, out_dim, batch), jnp.float32),
        scratch_types=[
            pltpu.VMEM((num_keys * out_dim,), jnp.float32),
            pltpu.VMEM((num_cards * per_worker,), jnp.int32),
            [pltpu.VMEM((out_dim, per_worker), jnp.float32)] * 2,
            [pltpu.SemaphoreType.DMA] * 2,
        ],
    )
    def sc_embed(keys_hbm, table_hbm, out_hbm, tab_v, keys_v, block_v, sem_s):
        wid = lax.axis_index("s") * 2 + lax.axis_index("c")
        b0 = wid * per_worker

        pltpu.sync_copy(table_hbm, tab_v)
        for c in range(num_cards):
            pltpu.sync_copy(
                keys_hbm.at[pl.ds(c * batch + b0, per_worker)],
                keys_v.at[pl.ds(c * per_worker, per_worker)],
            )

        scatters = [None, None]
        for c in range(num_cards):
            bb = c % 2
            if c >= 2:
                scatters[bb].wait()

            def fill(i, carry, c=c, bb=bb):
                kvec = keys_v[pl.ds(c * per_worker + i * LANES, LANES)]
                for f in range(out_dim):
                    block_v[bb][f, pl.ds(i * LANES, LANES)] = plsc.load_gather(
                        tab_v, [kvec + (f * num_keys)])
                return carry

            lax.fori_loop(0, per_worker // LANES, fill, 0)
            scatters[bb] = pltpu.async_copy(
                block_v[bb],
                out_hbm.at[c, :, pl.ds(b0, per_worker)],
                sem_s[bb],
            )
        scatters[0].wait()
        scatters[1].wait()

    return sc_embed


_sc_embed_cached = None


def _get_sc_embed(batch, num_cards, out_dim, num_keys):
    global _sc_embed_cached
    if _sc_embed_cached is None:
        _sc_embed_cached = _make_sc_embed(batch, num_cards, out_dim, num_keys)
    return _sc_embed_cached


def kernel(ranks, suits, rank_weight, suit_weight):
    batch, num_cards = ranks.shape
    num_ranks, rank_dim = rank_weight.shape
    num_suits, suit_dim = suit_weight.shape
    out_dim = rank_dim + suit_dim
    num_keys = num_ranks * num_suits

    combined = jnp.concatenate(
        [
            jnp.repeat(rank_weight, num_suits, axis=0),
            jnp.tile(suit_weight, (num_ranks, 1)),
        ],
        axis=1,
    )
    table_t = combined.T.reshape(-1)

    keys_t = (ranks * num_suits + suits).T.reshape(-1)

    fn = _get_sc_embed(batch, num_cards, out_dim, num_keys)
    out_phys = fn(keys_t, table_t)
    return jnp.transpose(out_phys, (2, 0, 1))


# speedup vs baseline: 32.3439x; 1.6731x over previous
import functools

import jax
import jax.numpy as jnp
from jax import lax
from jax.experimental import pallas as pl
from jax.experimental.pallas import tpu as pltpu
from jax.experimental.pallas import tpu_sc as plsc

NUM_WORKERS = 32
LANES = 16


def _make_sc_embed(batch, num_cards, out_dim, num_keys):
    assert batch % (NUM_WORKERS * LANES) == 0
    per_worker = batch // NUM_WORKERS
    mesh = plsc.VectorSubcoreMesh(core_axis_name="c", subcore_axis_name="s")

    @functools.partial(
        pl.kernel,
        mesh=mesh,
        compiler_params=pltpu.CompilerParams(
            use_tc_tiling_on_sc=False, needs_layout_passes=False),
        out_type=jax.ShapeDtypeStruct((num_cards, out_dim, batch), jnp.float32),
        scratch_types=[
            pltpu.VMEM((num_keys * out_dim,), jnp.float32),
            pltpu.VMEM((num_cards * per_worker,), jnp.int32),
            [pltpu.VMEM((out_dim, per_worker), jnp.float32)] * 2,
            [pltpu.SemaphoreType.DMA] * 2,
        ],
    )
    def sc_embed(keys_hbm, table_hbm, out_hbm, tab_v, keys_v, block_v, sem_s):
        wid = lax.axis_index("s") * 2 + lax.axis_index("c")
        b0 = wid * per_worker

        pltpu.sync_copy(table_hbm, tab_v)
        for c in range(num_cards):
            pltpu.sync_copy(
                keys_hbm.at[pl.ds(c * batch + b0, per_worker)],
                keys_v.at[pl.ds(c * per_worker, per_worker)],
            )

        scatters = [None, None]
        for c in range(num_cards):
            bb = c % 2
            if c >= 2:
                scatters[bb].wait()

            @plsc.parallel_loop(0, per_worker // LANES, unroll=2)
            def _fill(i, c=c, bb=bb):
                kvec = keys_v[pl.ds(c * per_worker + i * LANES, LANES)]
                vals = [plsc.load_gather(tab_v, [kvec + (f * num_keys)])
                        for f in range(out_dim)]
                for f in range(out_dim):
                    block_v[bb][f, pl.ds(i * LANES, LANES)] = vals[f]
            scatters[bb] = pltpu.async_copy(
                block_v[bb],
                out_hbm.at[c, :, pl.ds(b0, per_worker)],
                sem_s[bb],
            )
        scatters[0].wait()
        scatters[1].wait()

    return sc_embed


_sc_embed_cached = None


def _get_sc_embed(batch, num_cards, out_dim, num_keys):
    global _sc_embed_cached
    if _sc_embed_cached is None:
        _sc_embed_cached = _make_sc_embed(batch, num_cards, out_dim, num_keys)
    return _sc_embed_cached


def kernel(ranks, suits, rank_weight, suit_weight):
    batch, num_cards = ranks.shape
    num_ranks, rank_dim = rank_weight.shape
    num_suits, suit_dim = suit_weight.shape
    out_dim = rank_dim + suit_dim
    num_keys = num_ranks * num_suits

    combined = jnp.concatenate(
        [
            jnp.repeat(rank_weight, num_suits, axis=0),
            jnp.tile(suit_weight, (num_ranks, 1)),
        ],
        axis=1,
    )
    table_t = combined.T.reshape(-1)

    keys_t = (ranks * num_suits + suits).T.reshape(-1)

    fn = _get_sc_embed(batch, num_cards, out_dim, num_keys)
    out_phys = fn(keys_t, table_t)
    return jnp.transpose(out_phys, (2, 0, 1))


# speedup vs baseline: 54.8747x; 1.6966x over previous
import functools

import jax
import jax.numpy as jnp
from jax import lax
from jax.experimental import pallas as pl
from jax.experimental.pallas import tpu as pltpu
from jax.experimental.pallas import tpu_sc as plsc

NUM_WORKERS = 32
LANES = 16


def _make_sc_embed(batch, num_cards, out_dim, num_keys):
    assert batch % (NUM_WORKERS * LANES) == 0
    per_worker = batch // NUM_WORKERS
    mesh = plsc.VectorSubcoreMesh(core_axis_name="c", subcore_axis_name="s")

    f_tiles = out_dim // 8
    groups = batch // 128
    tile_words = 8 * 128
    run = (per_worker // 128) * tile_words

    @functools.partial(
        pl.kernel,
        mesh=mesh,
        compiler_params=pltpu.CompilerParams(
            use_tc_tiling_on_sc=False, needs_layout_passes=False),
        out_type=jax.ShapeDtypeStruct(
            (num_cards, f_tiles, groups * tile_words), jnp.float32),
        scratch_types=[
            pltpu.VMEM((num_keys * out_dim,), jnp.float32),
            pltpu.VMEM((num_cards * per_worker,), jnp.int32),
            [pltpu.VMEM((f_tiles, run), jnp.float32)] * 2,
            [pltpu.SemaphoreType.DMA] * 2,
        ],
    )
    def sc_embed(keys_hbm, table_hbm, out_hbm, tab_v, keys_v, block_v, sem_s):
        wid = lax.axis_index("s") * 2 + lax.axis_index("c")
        b0 = wid * per_worker

        pltpu.sync_copy(table_hbm, tab_v)
        for c in range(num_cards):
            pltpu.sync_copy(
                keys_hbm.at[pl.ds(c * batch + b0, per_worker)],
                keys_v.at[pl.ds(c * per_worker, per_worker)],
            )

        scatters = [None, None]
        for c in range(num_cards):
            bb = c % 2
            if c >= 2:
                scatters[bb].wait()

            @plsc.parallel_loop(0, per_worker // LANES, unroll=2)
            def _fill(i, c=c, bb=bb):
                kvec = keys_v[pl.ds(c * per_worker + i * LANES, LANES)]
                ib = (i // 8) * tile_words + (i % 8) * LANES
                vals = [plsc.load_gather(tab_v, [kvec + (f * num_keys)])
                        for f in range(out_dim)]
                for f in range(out_dim):
                    block_v[bb][f // 8, pl.ds(ib + (f % 8) * 128, LANES)] = (
                        vals[f])
            scatters[bb] = pltpu.async_copy(
                block_v[bb],
                out_hbm.at[c, :, pl.ds(wid * run, run)],
                sem_s[bb],
            )
        scatters[0].wait()
        scatters[1].wait()

    return sc_embed


_sc_embed_cached = None


def _get_sc_embed(batch, num_cards, out_dim, num_keys):
    global _sc_embed_cached
    if _sc_embed_cached is None:
        _sc_embed_cached = _make_sc_embed(batch, num_cards, out_dim, num_keys)
    return _sc_embed_cached


def kernel(ranks, suits, rank_weight, suit_weight):
    batch, num_cards = ranks.shape
    num_ranks, rank_dim = rank_weight.shape
    num_suits, suit_dim = suit_weight.shape
    out_dim = rank_dim + suit_dim
    num_keys = num_ranks * num_suits

    combined = jnp.concatenate(
        [
            jnp.repeat(rank_weight, num_suits, axis=0),
            jnp.tile(suit_weight, (num_ranks, 1)),
        ],
        axis=1,
    )
    table_t = combined.T.reshape(-1)

    keys_t = (ranks * num_suits + suits).T.reshape(-1)

    fn = _get_sc_embed(batch, num_cards, out_dim, num_keys)
    out_tiled = fn(keys_t, table_t)
    out5 = out_tiled.reshape(num_cards, out_dim // 8, batch // 128, 8, 128)
    out = jnp.transpose(out5, (2, 4, 0, 1, 3))
    return out.reshape(batch, num_cards, out_dim)


# speedup vs baseline: 68.9024x; 1.2556x over previous
import functools

import jax
import jax.numpy as jnp
from jax import lax
from jax.experimental import pallas as pl
from jax.experimental.pallas import tpu as pltpu
from jax.experimental.pallas import tpu_sc as plsc

NUM_WORKERS = 32
LANES = 16


def _make_sc_embed(batch, num_cards, out_dim, num_keys):
    assert batch % (NUM_WORKERS * LANES) == 0
    per_worker = batch // NUM_WORKERS
    mesh = plsc.VectorSubcoreMesh(core_axis_name="c", subcore_axis_name="s")

    f_tiles = out_dim // 8
    groups = batch // 128
    tile_words = 8 * 128
    run = (per_worker // 128) * tile_words

    @functools.partial(
        pl.kernel,
        mesh=mesh,
        compiler_params=pltpu.CompilerParams(
            use_tc_tiling_on_sc=False, needs_layout_passes=False),
        out_type=jax.ShapeDtypeStruct(
            (num_cards, f_tiles, groups * tile_words), jnp.float32),
        scratch_types=[
            pltpu.VMEM((num_keys * out_dim,), jnp.float32),
            pltpu.VMEM((num_cards, per_worker), jnp.int32),
            [pltpu.VMEM((f_tiles, run), jnp.float32)] * 2,
            [pltpu.SemaphoreType.DMA] * 2,
        ],
    )
    def sc_embed(keys_hbm, table_hbm, out_hbm, tab_v, keys_v, block_v, sem_s):
        wid = lax.axis_index("s") * 2 + lax.axis_index("c")
        b0 = wid * per_worker

        pltpu.sync_copy(table_hbm, tab_v)
        pltpu.sync_copy(keys_hbm.at[:, pl.ds(b0, per_worker)], keys_v)

        scatters = [None, None]
        for c in range(num_cards):
            bb = c % 2
            if c >= 2:
                scatters[bb].wait()

            @plsc.parallel_loop(0, per_worker // LANES, unroll=2)
            def _fill(i, c=c, bb=bb):
                kvec = keys_v[c, pl.ds(i * LANES, LANES)]
                ib = (i // 8) * tile_words + (i % 8) * LANES
                vals = [plsc.load_gather(tab_v, [kvec + (f * num_keys)])
                        for f in range(out_dim)]
                for f in range(out_dim):
                    block_v[bb][f // 8, pl.ds(ib + (f % 8) * 128, LANES)] = (
                        vals[f])
            scatters[bb] = pltpu.async_copy(
                block_v[bb],
                out_hbm.at[c, :, pl.ds(wid * run, run)],
                sem_s[bb],
            )
        scatters[0].wait()
        scatters[1].wait()

    return sc_embed


_sc_embed_cached = None


def _get_sc_embed(batch, num_cards, out_dim, num_keys):
    global _sc_embed_cached
    if _sc_embed_cached is None:
        _sc_embed_cached = _make_sc_embed(batch, num_cards, out_dim, num_keys)
    return _sc_embed_cached


def kernel(ranks, suits, rank_weight, suit_weight):
    batch, num_cards = ranks.shape
    num_ranks, rank_dim = rank_weight.shape
    num_suits, suit_dim = suit_weight.shape
    out_dim = rank_dim + suit_dim
    num_keys = num_ranks * num_suits

    combined = jnp.concatenate(
        [
            jnp.repeat(rank_weight, num_suits, axis=0),
            jnp.tile(suit_weight, (num_ranks, 1)),
        ],
        axis=1,
    )
    table_t = combined.T.reshape(-1)

    keys_t = (ranks * num_suits + suits).T

    fn = _get_sc_embed(batch, num_cards, out_dim, num_keys)
    out_tiled = fn(keys_t, table_t)
    out5 = out_tiled.reshape(num_cards, out_dim // 8, batch // 128, 8, 128)
    out = jnp.transpose(out5, (2, 4, 0, 1, 3))
    return out.reshape(batch, num_cards, out_dim)


# speedup vs baseline: 74.4192x; 1.0801x over previous
import functools

import jax
import jax.numpy as jnp
from jax import lax
from jax.experimental import pallas as pl
from jax.experimental.pallas import tpu as pltpu
from jax.experimental.pallas import tpu_sc as plsc

NUM_WORKERS = 32
LANES = 16


def _make_sc_embed(batch, num_cards, out_dim, num_keys):
    assert batch % (NUM_WORKERS * LANES) == 0
    per_worker = batch // NUM_WORKERS
    mesh = plsc.VectorSubcoreMesh(core_axis_name="c", subcore_axis_name="s")

    f_tiles = out_dim // 8
    groups = batch // 128
    tile_words = 8 * 128
    run = (per_worker // 128) * tile_words

    @functools.partial(
        pl.kernel,
        mesh=mesh,
        compiler_params=pltpu.CompilerParams(
            use_tc_tiling_on_sc=False, needs_layout_passes=False),
        out_type=jax.ShapeDtypeStruct(
            (num_cards, f_tiles, groups * tile_words), jnp.float32),
        scratch_types=[
            pltpu.VMEM((num_keys * out_dim,), jnp.float32),
            pltpu.VMEM((num_cards, per_worker), jnp.int32),
            [pltpu.VMEM((f_tiles, run), jnp.float32)] * 2,
            [pltpu.SemaphoreType.DMA] * 2,
        ],
    )
    def sc_embed(keys_hbm, table_hbm, out_hbm, tab_v, keys_v, block_v, sem_s):
        wid = lax.axis_index("s") * 2 + lax.axis_index("c")
        b0 = wid * per_worker

        pltpu.sync_copy(table_hbm, tab_v)
        pltpu.sync_copy(keys_hbm.at[:, pl.ds(b0, per_worker)], keys_v)

        def do_card(c, bb, first):
            @pl.when(jnp.logical_not(first))
            def _():
                pltpu.make_async_copy(
                    block_v[bb],
                    out_hbm.at[c, :, pl.ds(wid * run, run)],
                    sem_s[bb],
                ).wait()

            @plsc.parallel_loop(0, per_worker // LANES, unroll=2)
            def _fill(i):
                kvec = keys_v[c, pl.ds(i * LANES, LANES)]
                ib = (i // 8) * tile_words + (i % 8) * LANES
                vals = [plsc.load_gather(tab_v, [kvec + (f * num_keys)])
                        for f in range(out_dim)]
                for f in range(out_dim):
                    block_v[bb][f // 8, pl.ds(ib + (f % 8) * 128, LANES)] = (
                        vals[f])

            pltpu.async_copy(
                block_v[bb],
                out_hbm.at[c, :, pl.ds(wid * run, run)],
                sem_s[bb],
            )

        def pair_body(t, carry):
            do_card(2 * t, 0, t == 0)
            do_card(2 * t + 1, 1, t == 0)
            return carry

        lax.fori_loop(0, num_cards // 2, pair_body, 0)
        for bb in range(2):
            pltpu.make_async_copy(
                block_v[bb],
                out_hbm.at[0, :, pl.ds(wid * run, run)],
                sem_s[bb],
            ).wait()

    return sc_embed


_sc_embed_cached = None


def _get_sc_embed(batch, num_cards, out_dim, num_keys):
    global _sc_embed_cached
    if _sc_embed_cached is None:
        _sc_embed_cached = _make_sc_embed(batch, num_cards, out_dim, num_keys)
    return _sc_embed_cached


def kernel(ranks, suits, rank_weight, suit_weight):
    batch, num_cards = ranks.shape
    num_ranks, rank_dim = rank_weight.shape
    num_suits, suit_dim = suit_weight.shape
    out_dim = rank_dim + suit_dim
    num_keys = num_ranks * num_suits

    combined = jnp.concatenate(
        [
            jnp.repeat(rank_weight, num_suits, axis=0),
            jnp.tile(suit_weight, (num_ranks, 1)),
        ],
        axis=1,
    )
    table_t = combined.T.reshape(-1)

    keys_t = (ranks * num_suits + suits).T

    fn = _get_sc_embed(batch, num_cards, out_dim, num_keys)
    out_tiled = fn(keys_t, table_t)
    out5 = out_tiled.reshape(num_cards, out_dim // 8, batch // 128, 8, 128)
    out = jnp.transpose(out5, (2, 4, 0, 1, 3))
    return out.reshape(batch, num_cards, out_dim)
